# baseline (speedup 1.0000x reference)
import functools

import jax
import jax.numpy as jnp
from jax import lax
from jax.experimental import pallas as pl
from jax.experimental.pallas import tpu as pltpu
from jax.experimental.pallas import tpu_sc as plsc

NC = 2
NS = 16
NW = NC * NS
LANES = 16


def _make_pool(B, L, D, idx_chunk):
  assert L % idx_chunk == 0
  n_chunks = L // idx_chunk
  assert B % NW == 0
  b_per_w = B // NW
  n_vreg = D // LANES
  inv_l = 1.0 / float(L)

  mesh = plsc.VectorSubcoreMesh(
      core_axis_name="c", subcore_axis_name="s", num_cores=NC,
      num_subcores=NS)

  @functools.partial(
      pl.kernel,
      out_type=jax.ShapeDtypeStruct((B, D), jnp.float32),
      mesh=mesh,
      scratch_types=[
          pltpu.VMEM((n_chunks, idx_chunk), jnp.int32),
          pltpu.VMEM((L, D), jnp.float32),
          pltpu.VMEM((b_per_w, D), jnp.float32),
          pltpu.SemaphoreType.DMA,
      ],
      compiler_params=pltpu.CompilerParams(use_tc_tiling_on_sc=False),
  )
  def pool(idx_hbm, emb_hbm, out_hbm, idx_v, rows_v, out_v, sem):
    wid = lax.axis_index("s") * NC + lax.axis_index("c")
    base = wid * b_per_w

    def body(b, _):
      pltpu.sync_copy(idx_hbm.at[base + b], idx_v)
      copies = []
      for c in range(n_chunks):
        copies.append(pltpu.async_copy(
            emb_hbm.at[idx_v.at[c]],
            rows_v.at[pl.ds(c * idx_chunk, idx_chunk)], sem))
      for cp in copies:
        cp.wait()

      def row_body(j, acc):
        return tuple(
            acc[k] + rows_v[j, pl.ds(k * LANES, LANES)]
            for k in range(n_vreg))

      zero = jnp.zeros((LANES,), jnp.float32)
      acc = lax.fori_loop(0, L, row_body, (zero,) * n_vreg)
      for k in range(n_vreg):
        out_v[b, pl.ds(k * LANES, LANES)] = acc[k] * inv_l
      return 0

    lax.fori_loop(0, b_per_w, body, 0)
    pltpu.sync_copy(out_v, out_hbm.at[pl.ds(base, b_per_w)])

  return pool


def _head_body(x_ref, wms_ref, w2_ref, b2_ref, wc_ref, bc_ref, o_ref):
  x = x_ref[...]
  dims = (((1,), (1,)), ((), ()))
  xp = lax.dot_general(x, wms_ref[...], dims,
                       preferred_element_type=jnp.float32)
  h = jnp.maximum(
      lax.dot_general(xp, w2_ref[...], dims,
                      preferred_element_type=jnp.float32) + b2_ref[...], 0.0)
  logits = lax.dot_general(h, wc_ref[...], dims,
                           preferred_element_type=jnp.float32) + bc_ref[...]
  m = jnp.max(logits, axis=1, keepdims=True)
  e = jnp.exp(logits - m)
  o_ref[...] = e / jnp.sum(e, axis=1, keepdims=True)


def kernel(indices, emb, W_ms, W_clf2, b_clf2, W_clf, b_clf):
  B, L = indices.shape
  V, D = emb.shape
  OUT_DIM = W_clf.shape[0]

  idx_chunk = 100
  idx3 = indices.astype(jnp.int32).reshape(B, L // idx_chunk, idx_chunk)
  pool = _make_pool(B, L, D, idx_chunk)
  x_ave = pool(idx3, emb)

  head = pl.pallas_call(
      _head_body,
      out_shape=jax.ShapeDtypeStruct((B, OUT_DIM), jnp.float32),
  )
  return head(x_ave, W_ms, W_clf2, b_clf2.reshape(1, D),
              W_clf, b_clf.reshape(1, OUT_DIM))


# speedup vs baseline: 1.2491x; 1.2491x over previous
import functools

import jax
import jax.numpy as jnp
from jax import lax
from jax.experimental import pallas as pl
from jax.experimental.pallas import tpu as pltpu
from jax.experimental.pallas import tpu_sc as plsc

NC = 2
NS = 16
NW = NC * NS
LANES = 16


def _make_pool(B, L, D, idx_chunk):
  assert L % idx_chunk == 0
  n_chunks = L // idx_chunk
  assert B % NW == 0
  b_per_w = B // NW
  n_vreg = D // LANES
  inv_l = 1.0 / float(L)

  mesh = plsc.VectorSubcoreMesh(
      core_axis_name="c", subcore_axis_name="s", num_cores=NC,
      num_subcores=NS)

  @functools.partial(
      pl.kernel,
      out_type=jax.ShapeDtypeStruct((B, D), jnp.float32),
      mesh=mesh,
      scratch_types=[
          pltpu.VMEM((b_per_w, n_chunks, idx_chunk), jnp.int32),
          pltpu.VMEM((2, L, D), jnp.float32),
          pltpu.VMEM((b_per_w, D), jnp.float32),
          pltpu.SemaphoreType.DMA,
          pltpu.SemaphoreType.DMA,
      ],
      compiler_params=pltpu.CompilerParams(use_tc_tiling_on_sc=False),
  )
  def pool(idx_hbm, emb_hbm, out_hbm, idx_v, rows_v, out_v, sem0, sem1):
    wid = lax.axis_index("s") * NC + lax.axis_index("c")
    base = wid * b_per_w
    sems = (sem0, sem1)

    pltpu.sync_copy(idx_hbm.at[pl.ds(base, b_per_w)], idx_v)

    def issue(b, buf):
      for c in range(n_chunks):
        pltpu.async_copy(
            emb_hbm.at[idx_v.at[b, c]],
            rows_v.at[buf, pl.ds(c * idx_chunk, idx_chunk)], sems[buf])

    def drain(b, buf):
      for c in range(n_chunks):
        pltpu.make_async_copy(
            emb_hbm.at[idx_v.at[b, c]],
            rows_v.at[buf, pl.ds(c * idx_chunk, idx_chunk)],
            sems[buf]).wait()

    def accumulate(b, buf):
      zero = jnp.zeros((LANES,), jnp.float32)

      @plsc.parallel_loop(0, L, 2, unroll=4, carry=(zero,) * (2 * n_vreg))
      def acc(j, a):
        return tuple(
            a[p * n_vreg + k] + rows_v[buf, j + p, pl.ds(k * LANES, LANES)]
            for p in range(2) for k in range(n_vreg))

      for k in range(n_vreg):
        out_v[b, pl.ds(k * LANES, LANES)] = (
            (acc[k] + acc[n_vreg + k]) * inv_l)

    issue(0, 0)

    def body(g, _):
      b0 = 2 * g
      issue(b0 + 1, 1)
      drain(b0, 0)
      accumulate(b0, 0)

      @pl.when(b0 + 2 < b_per_w)
      def _():
        issue(b0 + 2, 0)

      drain(b0 + 1, 1)
      accumulate(b0 + 1, 1)
      return 0

    lax.fori_loop(0, b_per_w // 2, body, 0)
    pltpu.sync_copy(out_v, out_hbm.at[pl.ds(base, b_per_w)])

  return pool


def _head_body(x_ref, wms_ref, w2_ref, b2_ref, wc_ref, bc_ref, o_ref):
  x = x_ref[...]
  dims = (((1,), (1,)), ((), ()))
  xp = lax.dot_general(x, wms_ref[...], dims,
                       preferred_element_type=jnp.float32)
  h = jnp.maximum(
      lax.dot_general(xp, w2_ref[...], dims,
                      preferred_element_type=jnp.float32) + b2_ref[...], 0.0)
  logits = lax.dot_general(h, wc_ref[...], dims,
                           preferred_element_type=jnp.float32) + bc_ref[...]
  m = jnp.max(logits, axis=1, keepdims=True)
  e = jnp.exp(logits - m)
  o_ref[...] = e / jnp.sum(e, axis=1, keepdims=True)


def kernel(indices, emb, W_ms, W_clf2, b_clf2, W_clf, b_clf):
  B, L = indices.shape
  V, D = emb.shape
  OUT_DIM = W_clf.shape[0]

  idx_chunk = 100
  idx3 = indices.astype(jnp.int32).reshape(B, L // idx_chunk, idx_chunk)
  pool = _make_pool(B, L, D, idx_chunk)
  x_ave = pool(idx3, emb)

  head = pl.pallas_call(
      _head_body,
      out_shape=jax.ShapeDtypeStruct((B, OUT_DIM), jnp.float32),
  )
  return head(x_ave, W_ms, W_clf2, b_clf2.reshape(1, D),
              W_clf, b_clf.reshape(1, OUT_DIM))


# speedup vs baseline: 1.2531x; 1.0032x over previous
import functools

import jax
import jax.numpy as jnp
from jax import lax
from jax.experimental import pallas as pl
from jax.experimental.pallas import tpu as pltpu
from jax.experimental.pallas import tpu_sc as plsc

NC = 2
NS = 16
NW = NC * NS
LANES = 16


def _make_pool(B, L, D, chunks):
  assert sum(chunks) == L
  offs = [sum(chunks[:i]) for i in range(len(chunks))]
  assert B % NW == 0
  b_per_w = B // NW
  n_vreg = D // LANES
  inv_l = 1.0 / float(L)

  mesh = plsc.VectorSubcoreMesh(
      core_axis_name="c", subcore_axis_name="s", num_cores=NC,
      num_subcores=NS)

  @functools.partial(
      pl.kernel,
      out_type=jax.ShapeDtypeStruct((B, D), jnp.float32),
      mesh=mesh,
      scratch_types=[
          pltpu.VMEM((b_per_w, L), jnp.int32),
          pltpu.VMEM((2, L, D), jnp.float32),
          pltpu.VMEM((b_per_w, D), jnp.float32),
          pltpu.SemaphoreType.DMA,
          pltpu.SemaphoreType.DMA,
      ],
      compiler_params=pltpu.CompilerParams(use_tc_tiling_on_sc=False),
  )
  def pool(idx_hbm, emb_hbm, out_hbm, idx_v, rows_v, out_v, sem0, sem1):
    wid = lax.axis_index("s") * NC + lax.axis_index("c")
    base = wid * b_per_w
    sems = (sem0, sem1)

    pltpu.sync_copy(idx_hbm.at[pl.ds(base, b_per_w)], idx_v)

    def issue(b, buf):
      for o, w in zip(offs, chunks):
        pltpu.async_copy(
            emb_hbm.at[idx_v.at[b, pl.ds(o, w)]],
            rows_v.at[buf, pl.ds(o, w)], sems[buf])

    def drain(b, buf):
      for o, w in zip(offs, chunks):
        pltpu.make_async_copy(
            emb_hbm.at[idx_v.at[b, pl.ds(o, w)]],
            rows_v.at[buf, pl.ds(o, w)],
            sems[buf]).wait()

    def accumulate(b, buf):
      zero = jnp.zeros((LANES,), jnp.float32)

      @plsc.parallel_loop(0, L, 2, unroll=4, carry=(zero,) * (2 * n_vreg))
      def acc(j, a):
        return tuple(
            a[p * n_vreg + k] + rows_v[buf, j + p, pl.ds(k * LANES, LANES)]
            for p in range(2) for k in range(n_vreg))

      for k in range(n_vreg):
        out_v[b, pl.ds(k * LANES, LANES)] = (
            (acc[k] + acc[n_vreg + k]) * inv_l)

    issue(0, 0)

    def body(g, _):
      b0 = 2 * g
      issue(b0 + 1, 1)
      drain(b0, 0)
      accumulate(b0, 0)

      @pl.when(b0 + 2 < b_per_w)
      def _():
        issue(b0 + 2, 0)

      drain(b0 + 1, 1)
      accumulate(b0 + 1, 1)
      return 0

    lax.fori_loop(0, b_per_w // 2, body, 0)
    pltpu.sync_copy(out_v, out_hbm.at[pl.ds(base, b_per_w)])

  return pool


def _head_body(x_ref, wms_ref, w2_ref, b2_ref, wc_ref, bc_ref, o_ref):
  x = x_ref[...]
  dims = (((1,), (1,)), ((), ()))
  xp = lax.dot_general(x, wms_ref[...], dims,
                       preferred_element_type=jnp.float32)
  h = jnp.maximum(
      lax.dot_general(xp, w2_ref[...], dims,
                      preferred_element_type=jnp.float32) + b2_ref[...], 0.0)
  logits = lax.dot_general(h, wc_ref[...], dims,
                           preferred_element_type=jnp.float32) + bc_ref[...]
  m = jnp.max(logits, axis=1, keepdims=True)
  e = jnp.exp(logits - m)
  o_ref[...] = e / jnp.sum(e, axis=1, keepdims=True)


def kernel(indices, emb, W_ms, W_clf2, b_clf2, W_clf, b_clf):
  B, L = indices.shape
  V, D = emb.shape
  OUT_DIM = W_clf.shape[0]

  pool = _make_pool(B, L, D, (104, 96))
  x_ave = pool(indices.astype(jnp.int32), emb)

  head = pl.pallas_call(
      _head_body,
      out_shape=jax.ShapeDtypeStruct((B, OUT_DIM), jnp.float32),
  )
  return head(x_ave, W_ms, W_clf2, b_clf2.reshape(1, D),
              W_clf, b_clf.reshape(1, OUT_DIM))


# speedup vs baseline: 1.3748x; 1.0972x over previous
import functools

import jax
import jax.numpy as jnp
from jax import lax
from jax.experimental import pallas as pl
from jax.experimental.pallas import tpu as pltpu
from jax.experimental.pallas import tpu_sc as plsc

NC = 2
NS = 16
NW = NC * NS
LANES = 16


def _make_pool(B, L, D, chunks):
  assert sum(chunks) == L
  offs = [sum(chunks[:i]) for i in range(len(chunks))]
  assert B % NW == 0
  b_per_w = B // NW
  n_vreg = D // LANES
  inv_l = 1.0 / float(L)

  mesh = plsc.VectorSubcoreMesh(
      core_axis_name="c", subcore_axis_name="s", num_cores=NC,
      num_subcores=NS)

  @functools.partial(
      pl.kernel,
      out_type=jax.ShapeDtypeStruct((B, D), jnp.float32),
      mesh=mesh,
      scratch_types=[
          pltpu.VMEM((b_per_w, L), jnp.int32),
          pltpu.VMEM((2, L, D), jnp.float32),
          pltpu.VMEM((b_per_w, D), jnp.float32),
          pltpu.SemaphoreType.DMA,
          pltpu.SemaphoreType.DMA,
      ],
      compiler_params=pltpu.CompilerParams(use_tc_tiling_on_sc=False),
  )
  def pool(idx_hbm, emb_hbm, out_hbm, idx_v, rows_v, out_v, sem0, sem1):
    wid = lax.axis_index("s") * NC + lax.axis_index("c")
    base = wid * b_per_w
    sems = (sem0, sem1)

    pltpu.sync_copy(idx_hbm.at[pl.ds(base, b_per_w)], idx_v)

    def issue(b, buf):
      for o, w in zip(offs, chunks):
        pltpu.async_copy(
            emb_hbm.at[idx_v.at[b, pl.ds(o, w)]],
            rows_v.at[buf, pl.ds(o, w)], sems[buf])

    def drain(b, buf):
      for o, w in zip(offs, chunks):
        pltpu.make_async_copy(
            emb_hbm.at[idx_v.at[b, pl.ds(o, w)]],
            rows_v.at[buf, pl.ds(o, w)],
            sems[buf]).wait()

    def accumulate(b, buf):
      zero = jnp.zeros((LANES,), jnp.float32)

      @plsc.parallel_loop(0, L, 2, unroll=4, carry=(zero,) * (2 * n_vreg))
      def acc(j, a):
        return tuple(
            a[p * n_vreg + k] + rows_v[buf, j + p, pl.ds(k * LANES, LANES)]
            for p in range(2) for k in range(n_vreg))

      for k in range(n_vreg):
        out_v[b, pl.ds(k * LANES, LANES)] = (
            (acc[k] + acc[n_vreg + k]) * inv_l)

    issue(0, 0)

    def body(g, _):
      b0 = 2 * g
      issue(b0 + 1, 1)
      drain(b0, 0)
      accumulate(b0, 0)

      @pl.when(b0 + 2 < b_per_w)
      def _():
        issue(b0 + 2, 0)

      drain(b0 + 1, 1)
      accumulate(b0 + 1, 1)
      return 0

    lax.fori_loop(0, b_per_w // 2, body, 0)
    pltpu.sync_copy(out_v, out_hbm.at[pl.ds(base, b_per_w)])

  return pool


def _head_body(x_ref, wms_ref, w2_ref, b2_ref, wc_ref, bc_ref, o_ref):
  x = x_ref[...]
  dims = (((1,), (1,)), ((), ()))
  xp = lax.dot_general(x, wms_ref[...], dims,
                       preferred_element_type=jnp.float32)
  h = jnp.maximum(
      lax.dot_general(xp, w2_ref[...], dims,
                      preferred_element_type=jnp.float32) + b2_ref[...], 0.0)
  logits = lax.dot_general(h, wc_ref[...], dims,
                           preferred_element_type=jnp.float32) + bc_ref[...]
  m = jnp.max(logits, axis=1, keepdims=True)
  e = jnp.exp(logits - m)
  o_ref[...] = e / jnp.sum(e, axis=1, keepdims=True)


def kernel(indices, emb, W_ms, W_clf2, b_clf2, W_clf, b_clf):
  B, L = indices.shape
  V, D = emb.shape
  OUT_DIM = W_clf.shape[0]

  emb2 = jnp.pad(emb, ((0, 0), (0, D))).reshape(2 * V, D)
  pool = _make_pool(B, L, D, (104, 96))
  x_ave = pool(indices.astype(jnp.int32) * 2, emb2)

  head = pl.pallas_call(
      _head_body,
      out_shape=jax.ShapeDtypeStruct((B, OUT_DIM), jnp.float32),
  )
  return head(x_ave, W_ms, W_clf2, b_clf2.reshape(1, D),
              W_clf, b_clf.reshape(1, OUT_DIM))


# speedup vs baseline: 1.8905x; 1.3751x over previous
import functools

import jax
import jax.numpy as jnp
from jax import lax
from jax.experimental import pallas as pl
from jax.experimental.pallas import tpu as pltpu
from jax.experimental.pallas import tpu_sc as plsc

NC = 2
NS = 16
NW = NC * NS
LANES = 16


def _make_pool(B, L, D, chunks):
  assert sum(chunks) == L
  offs = [sum(chunks[:i]) for i in range(len(chunks))]
  assert B % NW == 0
  b_per_w = B // NW
  n_vreg = D // LANES
  inv_l = 1.0 / float(L)

  mesh = plsc.VectorSubcoreMesh(
      core_axis_name="c", subcore_axis_name="s", num_cores=NC,
      num_subcores=NS)

  @functools.partial(
      pl.kernel,
      out_type=jax.ShapeDtypeStruct((B, D), jnp.float32),
      mesh=mesh,
      scratch_types=[
          pltpu.VMEM((b_per_w, L), jnp.int32),
          pltpu.VMEM((2, L, D), jnp.float32),
          pltpu.VMEM((b_per_w, D), jnp.float32),
          pltpu.SemaphoreType.DMA,
          pltpu.SemaphoreType.DMA,
      ],
      compiler_params=pltpu.CompilerParams(use_tc_tiling_on_sc=False),
  )
  def pool(idx_hbm, emb_hbm, out_hbm, idx_v, rows_v, out_v, sem0, sem1):
    wid = lax.axis_index("s") * NC + lax.axis_index("c")
    base = wid * b_per_w
    sems = (sem0, sem1)

    pltpu.sync_copy(idx_hbm.at[pl.ds(base, b_per_w)], idx_v)

    def issue(b, buf):
      for o, w in zip(offs, chunks):
        pltpu.async_copy(
            emb_hbm.at[idx_v.at[b, pl.ds(o, w)]],
            rows_v.at[buf, pl.ds(o, w)], sems[buf])

    def drain(b, buf):
      for o, w in zip(offs, chunks):
        pltpu.make_async_copy(
            emb_hbm.at[idx_v.at[b, pl.ds(o, w)]],
            rows_v.at[buf, pl.ds(o, w)],
            sems[buf]).wait()

    def accumulate(b, buf):
      zero = jnp.zeros((LANES,), jnp.float32)

      @plsc.parallel_loop(0, L, 2, unroll=4, carry=(zero,) * (2 * n_vreg))
      def acc(j, a):
        return tuple(
            a[p * n_vreg + k] + rows_v[buf, j + p, pl.ds(k * LANES, LANES)]
            for p in range(2) for k in range(n_vreg))

      for k in range(n_vreg):
        out_v[b, pl.ds(k * LANES, LANES)] = (
            (acc[k] + acc[n_vreg + k]) * inv_l)

    issue(0, 0)

    def body(g, _):
      b0 = 2 * g
      issue(b0 + 1, 1)
      drain(b0, 0)
      accumulate(b0, 0)

      @pl.when(b0 + 2 < b_per_w)
      def _():
        issue(b0 + 2, 0)

      drain(b0 + 1, 1)
      accumulate(b0 + 1, 1)
      return 0

    lax.fori_loop(0, b_per_w // 2, body, 0)
    pltpu.sync_copy(out_v, out_hbm.at[pl.ds(base, b_per_w)])

  return pool


def _transpose_body(in_ref, out_ref):
  out_ref[:, 0:in_ref.shape[0]] = in_ref[...].T


def _make_transpose(V, D, BK):
  grid = (V + BK - 1) // BK
  return pl.pallas_call(
      _transpose_body,
      grid=(grid,),
      in_specs=[pl.BlockSpec((D, BK), lambda j: (0, j))],
      out_specs=pl.BlockSpec((BK, 2 * D), lambda j: (j, 0)),
      out_shape=jax.ShapeDtypeStruct((V, 2 * D), jnp.float32),
  )


def _head_body(x_ref, wms_ref, w2_ref, b2_ref, wc_ref, bc_ref, o_ref):
  x = x_ref[...]
  dims = (((1,), (1,)), ((), ()))
  xp = lax.dot_general(x, wms_ref[...], dims,
                       preferred_element_type=jnp.float32)
  h = jnp.maximum(
      lax.dot_general(xp, w2_ref[...], dims,
                      preferred_element_type=jnp.float32) + b2_ref[...], 0.0)
  logits = lax.dot_general(h, wc_ref[...], dims,
                           preferred_element_type=jnp.float32) + bc_ref[...]
  m = jnp.max(logits, axis=1, keepdims=True)
  e = jnp.exp(logits - m)
  o_ref[...] = e / jnp.sum(e, axis=1, keepdims=True)


def kernel(indices, emb, W_ms, W_clf2, b_clf2, W_clf, b_clf):
  B, L = indices.shape
  V, D = emb.shape
  OUT_DIM = W_clf.shape[0]

  emb_rows = _make_transpose(V, D, 4096)(emb.T)
  emb2 = emb_rows.reshape(2 * V, D)
  pool = _make_pool(B, L, D, (104, 96))
  x_ave = pool(indices.astype(jnp.int32) * 2, emb2)

  head = pl.pallas_call(
      _head_body,
      out_shape=jax.ShapeDtypeStruct((B, OUT_DIM), jnp.float32),
  )
  return head(x_ave, W_ms, W_clf2, b_clf2.reshape(1, D),
              W_clf, b_clf.reshape(1, OUT_DIM))


# speedup vs baseline: 2.1230x; 1.1230x over previous
import functools

import jax
import jax.numpy as jnp
from jax import lax
from jax.experimental import pallas as pl
from jax.experimental.pallas import tpu as pltpu
from jax.experimental.pallas import tpu_sc as plsc

NC = 2
NS = 16
NW = NC * NS
LANES = 16
NBUF = 4


def _transpose_pack_body(i0, i1, i2, i3, out_ref):
  for g, in_ref in enumerate((i0, i1, i2, i3)):
    x = in_ref[...]
    xt = x.T
    bk, d = xt.shape
    xi = jax.lax.bitcast_convert_type(xt, jnp.int32)
    rnd = lax.bitwise_and(lax.shift_right_logical(xi, 16), 1) + 0x7FFF
    b16 = lax.shift_right_logical(xi + rnd, 16)
    packed = lax.bitwise_or(b16[:, :d // 2],
                            lax.shift_left(b16[:, d // 2:], 16))
    out_ref[:, g * (d // 2):(g + 1) * (d // 2)] = packed


def _make_stage(V, D, Q, BK):
  grid = Q // BK
  qb = Q // BK
  last = (V - 1) // BK
  in_specs = [
      pl.BlockSpec(
          (D, BK),
          functools.partial(
              lambda g, j: (0, jnp.minimum(g * qb + j, last)), g))
      for g in range(4)
  ]
  return pl.pallas_call(
      _transpose_pack_body,
      grid=(grid,),
      in_specs=in_specs,
      out_specs=pl.BlockSpec((BK, 2 * D), lambda j: (j, 0)),
      out_shape=jax.ShapeDtypeStruct((Q, 2 * D), jnp.int32),
  )


def _make_pool(B, L, D, chunks):
  assert sum(chunks) == L
  offs = [sum(chunks[:i]) for i in range(len(chunks))]
  assert B % NW == 0
  b_per_w = B // NW
  w32 = D // 2
  n_vld = w32 // LANES
  inv_l = 1.0 / float(L)

  mesh = plsc.VectorSubcoreMesh(
      core_axis_name="c", subcore_axis_name="s", num_cores=NC,
      num_subcores=NS)

  @functools.partial(
      pl.kernel,
      out_type=jax.ShapeDtypeStruct((B, D), jnp.float32),
      mesh=mesh,
      scratch_types=[
          pltpu.VMEM((b_per_w, L), jnp.int32),
          pltpu.VMEM((NBUF, L, w32), jnp.int32),
          pltpu.VMEM((b_per_w, D), jnp.float32),
      ] + [pltpu.SemaphoreType.DMA] * NBUF,
      compiler_params=pltpu.CompilerParams(use_tc_tiling_on_sc=False),
  )
  def pool(idx_hbm, emb_hbm, out_hbm, idx_v, rows_v, out_v, *sems):
    wid = lax.axis_index("s") * NC + lax.axis_index("c")
    base = wid * b_per_w

    pltpu.sync_copy(idx_hbm.at[pl.ds(base, b_per_w)], idx_v)

    def issue(b, buf):
      for o, w in zip(offs, chunks):
        pltpu.async_copy(
            emb_hbm.at[idx_v.at[b, pl.ds(o, w)]],
            rows_v.at[buf, pl.ds(o, w)], sems[buf])

    def drain(b, buf):
      for o, w in zip(offs, chunks):
        pltpu.make_async_copy(
            emb_hbm.at[idx_v.at[b, pl.ds(o, w)]],
            rows_v.at[buf, pl.ds(o, w)],
            sems[buf]).wait()

    def accumulate(b, buf):
      zero = jnp.zeros((LANES,), jnp.float32)

      @plsc.parallel_loop(0, L, 2, unroll=4, carry=(zero,) * (4 * n_vld))
      def acc(j, a):
        out = []
        for p in range(2):
          for q in range(n_vld):
            v = rows_v[buf, j + p, pl.ds(q * LANES, LANES)]
            lo = lax.bitcast_convert_type(
                lax.shift_left(v, 16), jnp.float32)
            hi = lax.bitcast_convert_type(
                lax.shift_left(lax.shift_right_logical(v, 16), 16),
                jnp.float32)
            k = (p * n_vld + q) * 2
            out.append(a[k] + lo)
            out.append(a[k + 1] + hi)
        return tuple(out)

      for q in range(n_vld):
        for h in range(2):
          val = (acc[q * 2 + h] + acc[(n_vld + q) * 2 + h]) * inv_l
          out_v[b, pl.ds(h * (D // 2) + q * LANES, LANES)] = val

    for b in range(NBUF):
      issue(b, b)

    def body(g, _):
      b0 = g * NBUF
      for ph in range(NBUF):
        b = b0 + ph
        drain(b, ph)
        accumulate(b, ph)

        @pl.when(b + NBUF < b_per_w)
        def _():
          issue(b + NBUF, ph)
      return 0

    lax.fori_loop(0, b_per_w // NBUF, body, 0)
    pltpu.sync_copy(out_v, out_hbm.at[pl.ds(base, b_per_w)])

  return pool


def _head_body(x_ref, wms_ref, w2_ref, b2_ref, wc_ref, bc_ref, o_ref):
  x = x_ref[...]
  dims = (((1,), (1,)), ((), ()))
  xp = lax.dot_general(x, wms_ref[...], dims,
                       preferred_element_type=jnp.float32)
  h = jnp.maximum(
      lax.dot_general(xp, w2_ref[...], dims,
                      preferred_element_type=jnp.float32) + b2_ref[...], 0.0)
  logits = lax.dot_general(h, wc_ref[...], dims,
                           preferred_element_type=jnp.float32) + bc_ref[...]
  m = jnp.max(logits, axis=1, keepdims=True)
  e = jnp.exp(logits - m)
  o_ref[...] = e / jnp.sum(e, axis=1, keepdims=True)


def kernel(indices, emb, W_ms, W_clf2, b_clf2, W_clf, b_clf):
  B, L = indices.shape
  V, D = emb.shape
  OUT_DIM = W_clf.shape[0]

  Q = 256000
  embT = emb.T
  packed = _make_stage(V, D, Q, 2048)(embT, embT, embT, embT)
  table = packed.reshape(4 * Q, D // 2)
  idx = indices.astype(jnp.int32)
  quarter = ((idx >= Q).astype(jnp.int32) + (idx >= 2 * Q).astype(jnp.int32)
             + (idx >= 3 * Q).astype(jnp.int32))
  vidx = idx * 4 - quarter * (4 * Q - 1)
  pool = _make_pool(B, L, D, (104, 96))
  x_ave = pool(vidx, table)

  head = pl.pallas_call(
      _head_body,
      out_shape=jax.ShapeDtypeStruct((B, OUT_DIM), jnp.float32),
  )
  return head(x_ave, W_ms, W_clf2, b_clf2.reshape(1, D),
              W_clf, b_clf.reshape(1, OUT_DIM))


# speedup vs baseline: 3.6367x; 1.7129x over previous
import functools

import jax
import jax.numpy as jnp
from jax import lax
from jax.experimental import pallas as pl
from jax.experimental.pallas import tpu as pltpu
from jax.experimental.pallas import tpu_sc as plsc

NC = 2
NS = 16
NW = NC * NS
LANES = 16
NBUF = 8
IDXW = 256


def _stage_body(Q, L, i0, i1, i2, i3, idx_ref, out_ref, vidx_ref):
  parts = []
  for in_ref in (i0, i1, i2, i3):
    x = in_ref[...]
    d = x.shape[0]
    xi = jax.lax.bitcast_convert_type(x, jnp.int32)
    rnd = lax.bitwise_and(lax.shift_right_logical(xi, 16), 1) + 0x7FFF
    b16 = lax.shift_right_logical(xi + rnd, 16)
    parts.append(lax.bitwise_or(b16[:d // 2, :],
                                lax.shift_left(b16[d // 2:, :], 16)))
  out_ref[...] = jnp.concatenate(parts, axis=0).T

  @pl.when(pl.program_id(0) == 0)
  def _():
    t = idx_ref[...]
    t = jnp.concatenate([t, t[:IDXW - L, :]], axis=0)
    ti = t.T
    q = ((ti >= Q).astype(jnp.int32) + (ti >= 2 * Q).astype(jnp.int32)
         + (ti >= 3 * Q).astype(jnp.int32))
    vidx_ref[...] = ti * 4 - q * (4 * Q - 1)


def _make_stage(V, D, B, L, Q, BK):
  grid = Q // BK
  qb = Q // BK
  last = (V - 1) // BK
  in_specs = [
      pl.BlockSpec(
          (D, BK),
          functools.partial(
              lambda g, j: (0, jnp.minimum(g * qb + j, last)), g))
      for g in range(4)
  ] + [pl.BlockSpec((L, B), lambda j: (0, 0))]
  return pl.pallas_call(
      functools.partial(_stage_body, Q, L),
      grid=(grid,),
      in_specs=in_specs,
      out_specs=[
          pl.BlockSpec((BK, 2 * D), lambda j: (j, 0)),
          pl.BlockSpec((B, IDXW), lambda j: (0, 0)),
      ],
      out_shape=[
          jax.ShapeDtypeStruct((Q, 2 * D), jnp.int32),
          jax.ShapeDtypeStruct((B, IDXW), jnp.int32),
      ],
  )


def _make_pool(B, L, D, chunks):
  assert sum(chunks) == L
  offs = [sum(chunks[:i]) for i in range(len(chunks))]
  assert B % NW == 0
  b_per_w = B // NW
  w32 = D // 2
  n_vld = w32 // LANES
  inv_l = 1.0 / float(L)

  mesh = plsc.VectorSubcoreMesh(
      core_axis_name="c", subcore_axis_name="s", num_cores=NC,
      num_subcores=NS)

  @functools.partial(
      pl.kernel,
      out_type=jax.ShapeDtypeStruct((B, 2 * D), jnp.float32),
      mesh=mesh,
      scratch_types=[
          pltpu.VMEM((b_per_w, IDXW), jnp.int32),
          pltpu.VMEM((NBUF, L, w32), jnp.int32),
          pltpu.VMEM((b_per_w, 2 * D), jnp.float32),
      ] + [pltpu.SemaphoreType.DMA] * NBUF,
      compiler_params=pltpu.CompilerParams(use_tc_tiling_on_sc=False),
  )
  def pool(idx_hbm, emb_hbm, out_hbm, idx_v, rows_v, out_v, *sems):
    wid = lax.axis_index("s") * NC + lax.axis_index("c")
    base = wid * b_per_w

    pltpu.sync_copy(idx_hbm.at[pl.ds(base, b_per_w)], idx_v)

    def issue(b, buf):
      for o, w in zip(offs, chunks):
        pltpu.async_copy(
            emb_hbm.at[idx_v.at[b, pl.ds(o, w)]],
            rows_v.at[buf, pl.ds(o, w)], sems[buf])

    def drain(b, buf):
      for o, w in zip(offs, chunks):
        pltpu.make_async_copy(
            emb_hbm.at[idx_v.at[b, pl.ds(o, w)]],
            rows_v.at[buf, pl.ds(o, w)],
            sems[buf]).wait()

    def accumulate(b, buf):
      zero = jnp.zeros((LANES,), jnp.float32)

      @plsc.parallel_loop(0, L, 2, unroll=4, carry=(zero,) * (4 * n_vld))
      def acc(j, a):
        out = []
        for p in range(2):
          for q in range(n_vld):
            v = rows_v[buf, j + p, pl.ds(q * LANES, LANES)]
            lo = lax.bitcast_convert_type(
                lax.shift_left(v, 16), jnp.float32)
            hi = lax.bitcast_convert_type(
                lax.shift_left(lax.shift_right_logical(v, 16), 16),
                jnp.float32)
            k = (p * n_vld + q) * 2
            out.append(a[k] + lo)
            out.append(a[k + 1] + hi)
        return tuple(out)

      for q in range(n_vld):
        for h in range(2):
          val = (acc[q * 2 + h] + acc[(n_vld + q) * 2 + h]) * inv_l
          out_v[b, pl.ds(h * (D // 2) + q * LANES, LANES)] = val

    for b in range(NBUF):
      issue(b, b)

    def body(g, _):
      b0 = g * NBUF
      for ph in range(NBUF):
        b = b0 + ph
        drain(b, ph)
        accumulate(b, ph)

        @pl.when(b + NBUF < b_per_w)
        def _():
          issue(b + NBUF, ph)
      return 0

    lax.fori_loop(0, b_per_w // NBUF, body, 0)
    pltpu.sync_copy(out_v, out_hbm.at[pl.ds(base, b_per_w)])

  return pool


def _head_body(x_ref, wms_ref, w2_ref, b2_ref, wc_ref, bc_ref, o_ref):
  x = x_ref[...][:, :wms_ref.shape[0]]
  dims = (((1,), (1,)), ((), ()))
  xp = lax.dot_general(x, wms_ref[...], dims,
                       preferred_element_type=jnp.float32)
  h = jnp.maximum(
      lax.dot_general(xp, w2_ref[...], dims,
                      preferred_element_type=jnp.float32) + b2_ref[...], 0.0)
  lt = lax.dot_general(wc_ref[...], h, dims,
                       preferred_element_type=jnp.float32) + bc_ref[...]
  m = jnp.max(lt, axis=0, keepdims=True)
  e = jnp.exp(lt - m)
  o_ref[...] = e / jnp.sum(e, axis=0, keepdims=True)


def kernel(indices, emb, W_ms, W_clf2, b_clf2, W_clf, b_clf):
  B, L = indices.shape
  V, D = emb.shape
  OUT_DIM = W_clf.shape[0]

  Q = 256000
  embT = emb.T
  idxT = indices.astype(jnp.int32).T
  packed, vidx = _make_stage(V, D, B, L, Q, 2048)(embT, embT, embT, embT,
                                                  idxT)
  table = packed.reshape(4 * Q, D // 2)
  pool = _make_pool(B, L, D, (104, 96))
  x_ave = pool(vidx, table)

  head = pl.pallas_call(
      _head_body,
      out_shape=jax.ShapeDtypeStruct((OUT_DIM, B), jnp.float32),
  )
  out_t = head(x_ave, W_ms, W_clf2, b_clf2.reshape(1, D),
               W_clf, b_clf.reshape(OUT_DIM, 1))
  return out_t.T
